# baseline (device time: 171522 ns/iter reference)
import functools

import jax
import jax.numpy as jnp
from jax import lax
from jax.experimental import pallas as pl
from jax.experimental.pallas import tpu as pltpu

N_DEV = 32
B = 2
SQ = 128
D = 512
HQ = 4
DH = 64
HD = HQ * DH
S_GLOBAL = N_DEV * SQ


def kernel(x, Wq, Wk, Wv, Wo):
    def body(x_ref, wq_ref, wk_ref, wv_ref, wo_ref, out_ref,
             kv_ref, send_sems, recv_sems):
        my = lax.axis_index("i")
        left = lax.rem(my + N_DEV - 1, N_DEV)
        right = lax.rem(my + 1, N_DEV)

        barrier = pltpu.get_barrier_semaphore()
        for nbr in (left, right):
            pl.semaphore_signal(barrier, inc=1, device_id=(nbr,),
                                device_id_type=pl.DeviceIdType.MESH)
        pl.semaphore_wait(barrier, 2)

        d_idx = lax.broadcasted_iota(jnp.int32, (SQ, HD), 1)
        half = lax.rem(d_idx, DH) // 2
        inv = jnp.exp(half.astype(jnp.float32) * (-2.0 / DH * jnp.log(10000.0)))
        posn = lax.broadcasted_iota(jnp.int32, (SQ, HD), 0) + my * SQ
        ang = posn.astype(jnp.float32) * inv
        cos_t = jnp.cos(ang)
        sin_t = jnp.sin(ang)
        even = lax.rem(d_idx, 2) == 0

        def rope(t):
            t_rot = jnp.where(even, -jnp.roll(t, -1, axis=1),
                              jnp.roll(t, 1, axis=1))
            return t * cos_t + t_rot * sin_t

        wq = wq_ref[...].astype(jnp.bfloat16)
        wk = wk_ref[...].astype(jnp.bfloat16)
        wv = wv_ref[...].astype(jnp.bfloat16)

        qs = []
        for b in range(B):
            xb = x_ref[b].astype(jnp.bfloat16)
            q = rope(jnp.dot(xb, wq, preferred_element_type=jnp.float32))
            k = rope(jnp.dot(xb, wk, preferred_element_type=jnp.float32))
            v = jnp.dot(xb, wv, preferred_element_type=jnp.float32)
            qs.append(q.astype(jnp.bfloat16))
            row = pl.ds(my * SQ, SQ)
            kv_ref[b, row, :HD] = k.astype(jnp.bfloat16)
            kv_ref[b, row, HD:] = v.astype(jnp.bfloat16)

        for h in range(N_DEV - 1):
            src = lax.rem(my + 2 * N_DEV - h, N_DEV)
            blk = pl.ds(src * SQ, SQ)
            rdma = pltpu.make_async_remote_copy(
                src_ref=kv_ref.at[:, blk, :],
                dst_ref=kv_ref.at[:, blk, :],
                send_sem=send_sems.at[h],
                recv_sem=recv_sems.at[h],
                device_id=(right,),
                device_id_type=pl.DeviceIdType.MESH,
            )
            rdma.start()
            rdma.wait()

        wo = wo_ref[...].astype(jnp.bfloat16)
        for b in range(B):
            ctx_heads = []
            for hh in range(HQ):
                qbh = qs[b][:, hh * DH:(hh + 1) * DH]
                kf = kv_ref[b, :, hh * DH:(hh + 1) * DH]
                vf = kv_ref[b, :, HD + hh * DH:HD + (hh + 1) * DH]
                s = lax.dot_general(
                    qbh, kf, (((1,), (1,)), ((), ())),
                    preferred_element_type=jnp.float32) * 0.125
                m = jnp.max(s, axis=1, keepdims=True)
                w = jnp.exp(s - m)
                w = w / jnp.sum(w, axis=1, keepdims=True)
                ctx = lax.dot_general(
                    w.astype(jnp.bfloat16), vf, (((1,), (0,)), ((), ())),
                    preferred_element_type=jnp.float32)
                ctx_heads.append(ctx)
            ctx_b = jnp.concatenate(ctx_heads, axis=1).astype(jnp.bfloat16)
            out_ref[b] = jnp.dot(ctx_b, wo, preferred_element_type=jnp.float32)

        @functools.partial(pl.run_scoped, sem=pltpu.SemaphoreType.REGULAR)
        def _(sem):
            for nbr in (left, right):
                pl.semaphore_signal(sem, inc=1, device_id=(nbr,),
                                    device_id_type=pl.DeviceIdType.MESH)
            pl.semaphore_wait(sem, 2)

    return pl.pallas_call(
        body,
        out_shape=jax.ShapeDtypeStruct((B, SQ, D), jnp.float32),
        in_specs=[pl.BlockSpec(memory_space=pltpu.VMEM)] * 5,
        out_specs=pl.BlockSpec(memory_space=pltpu.VMEM),
        scratch_shapes=[
            pltpu.VMEM((B, S_GLOBAL, 2 * HD), jnp.bfloat16),
            pltpu.SemaphoreType.DMA((N_DEV - 1,)),
            pltpu.SemaphoreType.DMA((N_DEV - 1,)),
        ],
        compiler_params=pltpu.CompilerParams(collective_id=0),
    )(x, Wq, Wk, Wv, Wo)


# device time: 135947 ns/iter; 1.2617x vs baseline; 1.2617x over previous
import functools

import jax
import jax.numpy as jnp
from jax import lax
from jax.experimental import pallas as pl
from jax.experimental.pallas import tpu as pltpu

N_DEV = 32
B = 2
SQ = 128
D = 512
HQ = 4
DH = 64
HD = HQ * DH
S_GLOBAL = N_DEV * SQ


def kernel(x, Wq, Wk, Wv, Wo):
    def body(x_ref, wq_ref, wk_ref, wv_ref, wo_ref, out_ref,
             kv_ref, cw_send, cw_recv, ccw_send, ccw_recv):
        my = lax.axis_index("i")
        left = lax.rem(my + N_DEV - 1, N_DEV)
        right = lax.rem(my + 1, N_DEV)

        barrier = pltpu.get_barrier_semaphore()
        for nbr in (left, right):
            pl.semaphore_signal(barrier, inc=1, device_id=(nbr,),
                                device_id_type=pl.DeviceIdType.MESH)
        pl.semaphore_wait(barrier, 2)

        d_idx = lax.broadcasted_iota(jnp.int32, (SQ, HD), 1)
        half = lax.rem(d_idx, DH) // 2
        inv = jnp.exp(half.astype(jnp.float32) * (-2.0 / DH * jnp.log(10000.0)))
        posn = lax.broadcasted_iota(jnp.int32, (SQ, HD), 0) + my * SQ
        ang = posn.astype(jnp.float32) * inv
        cos_t = jnp.cos(ang)
        sin_t = jnp.sin(ang)
        even = lax.rem(d_idx, 2) == 0

        def rope(t):
            t_rot = jnp.where(even, -jnp.roll(t, -1, axis=1),
                              jnp.roll(t, 1, axis=1))
            return t * cos_t + t_rot * sin_t

        wq = wq_ref[...].astype(jnp.bfloat16)
        wk = wk_ref[...].astype(jnp.bfloat16)
        wv = wv_ref[...].astype(jnp.bfloat16)

        qs = []
        for b in range(B):
            xb = x_ref[b].astype(jnp.bfloat16)
            q = rope(jnp.dot(xb, wq, preferred_element_type=jnp.float32))
            k = rope(jnp.dot(xb, wk, preferred_element_type=jnp.float32))
            v = jnp.dot(xb, wv, preferred_element_type=jnp.float32)
            qs.append(q.astype(jnp.bfloat16))
            row = pl.ds(my * SQ, SQ)
            kv_ref[b, row, :HD] = k.astype(jnp.bfloat16)
            kv_ref[b, row, HD:] = v.astype(jnp.bfloat16)

        for h in range(N_DEV // 2):
            src_cw = lax.rem(my + 2 * N_DEV - h, N_DEV)
            blk_cw = pl.ds(src_cw * SQ, SQ)
            rdma_cw = pltpu.make_async_remote_copy(
                src_ref=kv_ref.at[:, blk_cw, :],
                dst_ref=kv_ref.at[:, blk_cw, :],
                send_sem=cw_send.at[h],
                recv_sem=cw_recv.at[h],
                device_id=(right,),
                device_id_type=pl.DeviceIdType.MESH,
            )
            rdma_cw.start()
            if h < N_DEV // 2 - 1:
                src_ccw = lax.rem(my + h, N_DEV)
                blk_ccw = pl.ds(src_ccw * SQ, SQ)
                rdma_ccw = pltpu.make_async_remote_copy(
                    src_ref=kv_ref.at[:, blk_ccw, :],
                    dst_ref=kv_ref.at[:, blk_ccw, :],
                    send_sem=ccw_send.at[h],
                    recv_sem=ccw_recv.at[h],
                    device_id=(left,),
                    device_id_type=pl.DeviceIdType.MESH,
                )
                rdma_ccw.start()
                rdma_cw.wait()
                rdma_ccw.wait()
            else:
                rdma_cw.wait()

        wo = wo_ref[...].astype(jnp.bfloat16)
        for b in range(B):
            ctx_heads = []
            for hh in range(HQ):
                qbh = qs[b][:, hh * DH:(hh + 1) * DH]
                kf = kv_ref[b, :, hh * DH:(hh + 1) * DH]
                vf = kv_ref[b, :, HD + hh * DH:HD + (hh + 1) * DH]
                s = lax.dot_general(
                    qbh, kf, (((1,), (1,)), ((), ())),
                    preferred_element_type=jnp.float32) * 0.125
                m = jnp.max(s, axis=1, keepdims=True)
                w = jnp.exp(s - m)
                w = w / jnp.sum(w, axis=1, keepdims=True)
                ctx = lax.dot_general(
                    w.astype(jnp.bfloat16), vf, (((1,), (0,)), ((), ())),
                    preferred_element_type=jnp.float32)
                ctx_heads.append(ctx)
            ctx_b = jnp.concatenate(ctx_heads, axis=1).astype(jnp.bfloat16)
            out_ref[b] = jnp.dot(ctx_b, wo, preferred_element_type=jnp.float32)

        @functools.partial(pl.run_scoped, sem=pltpu.SemaphoreType.REGULAR)
        def _(sem):
            for nbr in (left, right):
                pl.semaphore_signal(sem, inc=1, device_id=(nbr,),
                                    device_id_type=pl.DeviceIdType.MESH)
            pl.semaphore_wait(sem, 2)

    return pl.pallas_call(
        body,
        out_shape=jax.ShapeDtypeStruct((B, SQ, D), jnp.float32),
        in_specs=[pl.BlockSpec(memory_space=pltpu.VMEM)] * 5,
        out_specs=pl.BlockSpec(memory_space=pltpu.VMEM),
        scratch_shapes=[
            pltpu.VMEM((B, S_GLOBAL, 2 * HD), jnp.bfloat16),
            pltpu.SemaphoreType.DMA((N_DEV // 2,)),
            pltpu.SemaphoreType.DMA((N_DEV // 2,)),
            pltpu.SemaphoreType.DMA((N_DEV // 2 - 1,)),
            pltpu.SemaphoreType.DMA((N_DEV // 2 - 1,)),
        ],
        compiler_params=pltpu.CompilerParams(collective_id=0),
    )(x, Wq, Wk, Wv, Wo)


# device time: 116560 ns/iter; 1.4715x vs baseline; 1.1663x over previous
import functools

import jax
import jax.numpy as jnp
from jax import lax
from jax.experimental import pallas as pl
from jax.experimental.pallas import tpu as pltpu

N_DEV = 32
B = 2
SQ = 128
D = 512
HQ = 4
DH = 64
HD = HQ * DH
S_GLOBAL = N_DEV * SQ
CW = N_DEV // 2
CCW = N_DEV // 2 - 1


def kernel(x, Wq, Wk, Wv, Wo):
    def body(x_ref, wq_ref, wk_ref, wv_ref, wo_ref, out_ref, kv_ref,
             kcw_s, kcw_r, vcw_s, vcw_r, kccw_s, kccw_r, vccw_s, vccw_r):
        my = lax.axis_index("i")
        left = lax.rem(my + N_DEV - 1, N_DEV)
        right = lax.rem(my + 1, N_DEV)

        barrier = pltpu.get_barrier_semaphore()
        for nbr in (left, right):
            pl.semaphore_signal(barrier, inc=1, device_id=(nbr,),
                                device_id_type=pl.DeviceIdType.MESH)
        pl.semaphore_wait(barrier, 2)

        d_idx = lax.broadcasted_iota(jnp.int32, (SQ, HD), 1)
        half = lax.rem(d_idx, DH) // 2
        inv = jnp.exp(half.astype(jnp.float32) * (-2.0 / DH * jnp.log(10000.0)))
        posn = lax.broadcasted_iota(jnp.int32, (SQ, HD), 0) + my * SQ
        ang = posn.astype(jnp.float32) * inv
        cos_t = jnp.cos(ang)
        sin_t = jnp.sin(ang)
        even = lax.rem(d_idx, 2) == 0

        def rope(t):
            t_rot = jnp.where(even, -jnp.roll(t, -1, axis=1),
                              jnp.roll(t, 1, axis=1))
            return t * cos_t + t_rot * sin_t

        wk = wk_ref[...].astype(jnp.bfloat16)
        wv = wv_ref[...].astype(jnp.bfloat16)
        for b in range(B):
            xb = x_ref[b].astype(jnp.bfloat16)
            k = rope(jnp.dot(xb, wk, preferred_element_type=jnp.float32))
            v = jnp.dot(xb, wv, preferred_element_type=jnp.float32)
            row = pl.ds(my * SQ, SQ)
            kv_ref[b, row, :HD] = k.astype(jnp.bfloat16)
            kv_ref[b, row, HD:] = v.astype(jnp.bfloat16)

        sent = []

        def start(send_arr, recv_arr, idx, delta, col_off, dev):
            blk = pl.ds(lax.rem(my + 2 * N_DEV + delta, N_DEV) * SQ, SQ)
            r = pltpu.make_async_remote_copy(
                src_ref=kv_ref.at[:, blk, pl.ds(col_off, HD)],
                dst_ref=kv_ref.at[:, blk, pl.ds(col_off, HD)],
                send_sem=send_arr.at[idx],
                recv_sem=recv_arr.at[idx],
                device_id=(dev,),
                device_id_type=pl.DeviceIdType.MESH,
            )
            r.start()
            sent.append(r)
            return r

        desc = {}
        desc["kcw", 0] = start(kcw_s, kcw_r, 0, 0, 0, right)
        desc["vcw", 0] = start(vcw_s, vcw_r, 0, 0, HD, right)
        desc["kccw", 0] = start(kccw_s, kccw_r, 0, 0, 0, left)
        desc["vccw", 0] = start(vccw_s, vccw_r, 0, 0, HD, left)

        wq = wq_ref[...].astype(jnp.bfloat16)
        qs = []
        for b in range(B):
            xb = x_ref[b].astype(jnp.bfloat16)
            q = rope(jnp.dot(xb, wq, preferred_element_type=jnp.float32))
            qs.append(q.astype(jnp.bfloat16))

        state = {}
        for b in range(B):
            for hh in range(HQ):
                state[b, hh] = (
                    jnp.full((SQ, 1), -1e30, jnp.float32),
                    jnp.zeros((SQ, 1), jnp.float32),
                    jnp.zeros((SQ, DH), jnp.float32),
                )

        def flash_update(delta):
            blk = pl.ds(lax.rem(my + 2 * N_DEV + delta, N_DEV) * SQ, SQ)
            for b in range(B):
                for hh in range(HQ):
                    m, l, acc = state[b, hh]
                    kb = kv_ref[b, blk, hh * DH:(hh + 1) * DH]
                    vb = kv_ref[b, blk, HD + hh * DH:HD + (hh + 1) * DH]
                    qbh = qs[b][:, hh * DH:(hh + 1) * DH]
                    s = lax.dot_general(
                        qbh, kb, (((1,), (1,)), ((), ())),
                        preferred_element_type=jnp.float32) * 0.125
                    m_new = jnp.maximum(m, jnp.max(s, axis=1, keepdims=True))
                    p = jnp.exp(s - m_new)
                    corr = jnp.exp(m - m_new)
                    pv = lax.dot_general(
                        p.astype(jnp.bfloat16), vb, (((1,), (0,)), ((), ())),
                        preferred_element_type=jnp.float32)
                    state[b, hh] = (
                        m_new,
                        l * corr + jnp.sum(p, axis=1, keepdims=True),
                        acc * corr + pv,
                    )

        flash_update(0)

        for h in range(CW):
            desc["kcw", h].wait_recv()
            if h + 1 < CW:
                desc["kcw", h + 1] = start(
                    kcw_s, kcw_r, h + 1, -(h + 1), 0, right)
            desc["vcw", h].wait_recv()
            if h + 1 < CW:
                desc["vcw", h + 1] = start(
                    vcw_s, vcw_r, h + 1, -(h + 1), HD, right)
            if h < CCW:
                desc["kccw", h].wait_recv()
                if h + 1 < CCW:
                    desc["kccw", h + 1] = start(
                        kccw_s, kccw_r, h + 1, h + 1, 0, left)
                desc["vccw", h].wait_recv()
                if h + 1 < CCW:
                    desc["vccw", h + 1] = start(
                        vccw_s, vccw_r, h + 1, h + 1, HD, left)
            flash_update(-(h + 1))
            if h < CCW:
                flash_update(h + 1)

        wo = wo_ref[...].astype(jnp.bfloat16)
        for b in range(B):
            ctx_b = jnp.concatenate(
                [state[b, hh][2] / state[b, hh][1] for hh in range(HQ)],
                axis=1).astype(jnp.bfloat16)
            out_ref[b] = jnp.dot(ctx_b, wo, preferred_element_type=jnp.float32)

        for r in sent:
            r.wait_send()

        @functools.partial(pl.run_scoped, sem=pltpu.SemaphoreType.REGULAR)
        def _(sem):
            for nbr in (left, right):
                pl.semaphore_signal(sem, inc=1, device_id=(nbr,),
                                    device_id_type=pl.DeviceIdType.MESH)
            pl.semaphore_wait(sem, 2)

    return pl.pallas_call(
        body,
        out_shape=jax.ShapeDtypeStruct((B, SQ, D), jnp.float32),
        in_specs=[pl.BlockSpec(memory_space=pltpu.VMEM)] * 5,
        out_specs=pl.BlockSpec(memory_space=pltpu.VMEM),
        scratch_shapes=[
            pltpu.VMEM((B, S_GLOBAL, 2 * HD), jnp.bfloat16),
            pltpu.SemaphoreType.DMA((CW,)),
            pltpu.SemaphoreType.DMA((CW,)),
            pltpu.SemaphoreType.DMA((CW,)),
            pltpu.SemaphoreType.DMA((CW,)),
            pltpu.SemaphoreType.DMA((CCW,)),
            pltpu.SemaphoreType.DMA((CCW,)),
            pltpu.SemaphoreType.DMA((CCW,)),
            pltpu.SemaphoreType.DMA((CCW,)),
        ],
        compiler_params=pltpu.CompilerParams(collective_id=0),
    )(x, Wq, Wk, Wv, Wo)
